# Initial kernel scaffold; baseline (speedup 1.0000x reference)
import jax
import jax.numpy as jnp
from jax.experimental import pallas as pl


def kernel(user_emb, item_emb, W_ui0, b_ui0, W_ui1, b_ui1, W_so0, b_so0, W_so1, b_so1, ln0_g, ln0_b, ln1_g, ln1_b, ln2_g, ln2_b, ln3_g, ln3_b, ui_edge_index, social_edge_index):
    raise NotImplementedError("write your pallas kernel here")


# speedup vs baseline: 2.6223x; 2.6223x over previous
import functools

import jax
import jax.numpy as jnp
from jax import lax
from jax.experimental import pallas as pl
from jax.experimental.pallas import tpu as pltpu
from jax.experimental.pallas import tpu_sc as plsc

N = 10000
D = 128
E_UI = 320000
E_SOC = 160000

NC, NS = 2, 16
NW = NC * NS
NPAD = 10240
RPT = NPAD // NS

C_UI = 80
NCH_UI = E_UI // (NW * C_UI)
C_SO = 40
NCH_SO = E_SOC // (NW * C_SO)


@functools.partial(
    pl.kernel,
    out_type=jax.ShapeDtypeStruct((6 * NPAD, D), jnp.float32),
    mesh=plsc.VectorSubcoreMesh(core_axis_name="c", subcore_axis_name="s"),
    scratch_types=[
        pltpu.VMEM_SHARED((NPAD, D), jnp.float32),
        pltpu.VMEM((C_UI,), jnp.int32),
        pltpu.VMEM((C_UI,), jnp.int32),
        pltpu.VMEM((C_UI, D), jnp.float32),
        pltpu.VMEM((C_SO,), jnp.int32),
        pltpu.VMEM((C_SO,), jnp.int32),
        pltpu.VMEM((C_SO, D), jnp.float32),
    ],
)
def _seg3(ci_h, cu_h, cs_h, ui_dst_h, ui_src_h, so_col_h, so_row_h, zeros_h,
          out_h, acc, gi_u, si_u, rows_u, gi_s, si_s, rows_s):
    cid = lax.axis_index("c")
    sid = lax.axis_index("s")
    wid = cid * NS + sid
    row0 = sid * RPT

    def zero_own_rows():
        pltpu.sync_copy(zeros_h, acc.at[pl.ds(row0, RPT)])

    def run_set(tbl_h, gidx_h, sidx_h, per_w, c, nch, gi, si, rows, set_id):
        base = wid * per_w

        def chunk(k, carry):
            off = pl.multiple_of(base + k * c, 8)
            pltpu.sync_copy(gidx_h.at[pl.ds(off, c)], gi)
            pltpu.sync_copy(sidx_h.at[pl.ds(off, c)], si)
            pltpu.sync_copy(tbl_h.at[gi], rows)
            pltpu.sync_copy(rows, acc.at[si], add=True)
            return carry

        lax.fori_loop(0, nch, chunk, 0)
        plsc.subcore_barrier()
        out_row = pl.multiple_of((2 * set_id + cid) * NPAD + row0, 8)
        pltpu.sync_copy(acc.at[pl.ds(row0, RPT)], out_h.at[pl.ds(out_row, RPT)])
        zero_own_rows()
        plsc.subcore_barrier()

    zero_own_rows()
    plsc.subcore_barrier()
    run_set(ci_h, ui_dst_h, ui_src_h, E_UI // NW, C_UI, NCH_UI,
            gi_u, si_u, rows_u, 0)
    run_set(cu_h, ui_src_h, ui_dst_h, E_UI // NW, C_UI, NCH_UI,
            gi_u, si_u, rows_u, 1)
    run_set(cs_h, so_col_h, so_row_h, E_SOC // NW, C_SO, NCH_SO,
            gi_s, si_s, rows_s, 2)


_R = 1000


def _dense_body(p_ref, res_ref, w_ref, b_ref, g_ref, bb_ref, o_ref):
    a = p_ref[0] + p_ref[1]
    h = jnp.dot(a, w_ref[...], preferred_element_type=jnp.float32) + b_ref[...]
    x = res_ref[...] + h
    m = jnp.mean(x, axis=-1, keepdims=True)
    v = jnp.mean((x - m) ** 2, axis=-1, keepdims=True)
    o_ref[...] = (x - m) / jnp.sqrt(v + 1e-5) * g_ref[...] + bb_ref[...]


def _dense(parts, res, w, b, g, bb):
    return pl.pallas_call(
        _dense_body,
        grid=(N // _R,),
        in_specs=[
            pl.BlockSpec((2, _R, D), lambda i: (0, i, 0)),
            pl.BlockSpec((_R, D), lambda i: (i, 0)),
            pl.BlockSpec((D, D), lambda i: (0, 0)),
            pl.BlockSpec((1, D), lambda i: (0, 0)),
            pl.BlockSpec((1, D), lambda i: (0, 0)),
            pl.BlockSpec((1, D), lambda i: (0, 0)),
        ],
        out_specs=pl.BlockSpec((_R, D), lambda i: (i, 0)),
        out_shape=jax.ShapeDtypeStruct((N, D), jnp.float32),
    )(parts, res, w, b.reshape(1, D), g.reshape(1, D), bb.reshape(1, D))


def kernel(user_emb, item_emb, W_ui0, b_ui0, W_ui1, b_ui1, W_so0, b_so0,
           W_so1, b_so1, ln0_g, ln0_b, ln1_g, ln1_b, ln2_g, ln2_b,
           ln3_g, ln3_b, ui_edge_index, social_edge_index):
    ui_src = ui_edge_index[0].astype(jnp.int32)
    ui_dst = ui_edge_index[1].astype(jnp.int32)
    so_row = social_edge_index[0].astype(jnp.int32)
    so_col = social_edge_index[1].astype(jnp.int32)
    zeros = jnp.zeros((RPT, D), jnp.float32)

    W_ui = [(W_ui0, b_ui0), (W_ui1, b_ui1)]
    W_so = [(W_so0, b_so0), (W_so1, b_so1)]
    lns = [(ln0_g, ln0_b), (ln1_g, ln1_b), (ln2_g, ln2_b), (ln3_g, ln3_b)]

    cu, ci, cs = user_emb, item_emb, user_emb
    ui_embs = [user_emb]
    so_embs = [user_emb]
    for i in range(2):
        parts = _seg3(ci, cu, cs, ui_dst, ui_src, so_col, so_row, zeros)
        parts = parts.reshape(3, 2, NPAD, D)
        wl, bl = W_ui[i]
        ws, bs = W_so[i]
        g0, c0 = lns[2 * i]
        g1, c1 = lns[2 * i + 1]
        cu = _dense(parts[0], cu, wl, bl, g0, c0)
        ci = _dense(parts[1], ci, wl, bl, g1, c1)
        cs = _dense(parts[2], cs, ws, bs, g0, c0)
        ui_embs.append(cu)
        so_embs.append(cs)
    return jnp.stack(ui_embs + so_embs + [ci])
